# baseline (speedup 1.0000x reference)
import functools

import jax
import jax.numpy as jnp
from jax import lax
from jax.experimental import pallas as pl
from jax.experimental.pallas import tpu as pltpu
from jax.experimental.pallas import tpu_sc as plsc

_NUM_WORKERS = 32
_CHUNK = 32768


def _sc_copy_body(x_hbm, o_hbm, buf, in_sems, out_sems):
    n_total = x_hbm.shape[0]
    per_worker = n_total // _NUM_WORKERS
    n_chunks = per_worker // _CHUNK
    wid = lax.axis_index("s") * 2 + lax.axis_index("c")
    base = wid * per_worker

    def in_copy(i, b):
        return pltpu.make_async_copy(
            x_hbm.at[pl.ds(base + i * _CHUNK, _CHUNK)], buf.at[b], in_sems.at[b]
        )

    def out_copy(i, b):
        return pltpu.make_async_copy(
            buf.at[b], o_hbm.at[pl.ds(base + i * _CHUNK, _CHUNK)], out_sems.at[b]
        )

    in_copy(0, 0).start()
    out_pending = [None, None]
    for i in range(n_chunks):
        b = i % 2
        nb = (i + 1) % 2
        if i + 1 < n_chunks:
            if out_pending[nb] is not None:
                out_pending[nb].wait()
                out_pending[nb] = None
            in_copy(i + 1, nb).start()
        in_copy(i, b).wait()
        c = out_copy(i, b)
        c.start()
        out_pending[b] = c
    for b in range(2):
        if out_pending[b] is not None:
            out_pending[b].wait()


def kernel(x):
    B, C, N = x.shape
    n_total = B * C * N
    x_flat = x.reshape(n_total)
    mesh = plsc.VectorSubcoreMesh(core_axis_name="c", subcore_axis_name="s")
    out = pl.kernel(
        _sc_copy_body,
        out_type=jax.ShapeDtypeStruct((n_total,), x.dtype),
        mesh=mesh,
        scratch_types=[
            pltpu.VMEM((2, _CHUNK), jnp.float32),
            pltpu.SemaphoreType.DMA((2,)),
            pltpu.SemaphoreType.DMA((2,)),
        ],
    )(x_flat)
    return out.reshape(B, C * 4, N // 4)
